# baseline (speedup 1.0000x reference)
import functools

import jax
import jax.numpy as jnp
from jax import lax
from jax.experimental import pallas as pl
from jax.experimental.pallas import tpu as pltpu
from jax.experimental.pallas import tpu_sc as plsc

D = 32
N_CHUNKS = 16
CHUNK = 512
EMB_OFF = 2
N_TOK = 16 * 1024
NSEG = 2
SEG = N_TOK // NSEG
T_BLK = 512
NC, NS = 2, 16
NW = NC * NS
TOK_PER_TILE = SEG // NW
SC_BLK = 128
N_SC_BLKS = TOK_PER_TILE // SC_BLK


def _tc_argmin_body(x_ref, cb_ref, idx_ref):
    x = x_ref[...]
    xn = jnp.sum(x * x, axis=1, keepdims=True)
    x2 = x + x
    cols = []
    for c in range(N_CHUNKS):
        cb = cb_ref[c * CHUNK:(c + 1) * CHUNK, :]
        cn = jnp.sum(cb * cb, axis=1)[None, :]
        s2 = lax.dot_general(x2, cb, (((1,), (1,)), ((), ())),
                             preferred_element_type=jnp.float32)
        dist = (xn + cn) - s2
        dmin = jnp.min(dist, axis=1, keepdims=True)
        ii = lax.broadcasted_iota(jnp.int32, (1, CHUNK), 1).astype(jnp.float32)
        first_min = jnp.min(jnp.where(dist <= dmin, ii, jnp.float32(CHUNK)),
                            axis=1).astype(jnp.int32)
        cols.append(first_min + (EMB_OFF + c * CHUNK))
    idx_ref[...] = jnp.stack(cols, axis=0)


def _make_tc_argmin(block_off):
    return pl.pallas_call(
        _tc_argmin_body,
        grid=(SEG // T_BLK,),
        in_specs=[
            pl.BlockSpec((T_BLK, D), lambda i: (i + block_off, 0)),
            pl.BlockSpec((N_CHUNKS * CHUNK, D), lambda i: (0, 0)),
        ],
        out_specs=pl.BlockSpec((N_CHUNKS, T_BLK), lambda i: (0, i)),
        out_shape=jax.ShapeDtypeStruct((N_CHUNKS, SEG), jnp.int32),
    )


def _sc_gather_body(tok_off, idx_hbm, x_hbm, mask_hbm, jvec_hbm, emb_hbm,
                    q_hbm, part_hbm,
                    idx_v, buf_v, x_v, mask_v, out_v, jv_v, lacc_v, sem):
    cid = lax.axis_index("c")
    sid = lax.axis_index("s")
    wid = sid * NC + cid
    base = wid * TOK_PER_TILE

    pltpu.sync_copy(jvec_hbm, jv_v)
    jval = jv_v[...]
    lacc = jnp.zeros((16,), jnp.float32)

    for b in range(N_SC_BLKS):
        tok0 = base + b * SC_BLK
        glob0 = (tok_off + tok0) * D
        pltpu.sync_copy(idx_hbm.at[:, pl.ds(tok0, SC_BLK)], idx_v)
        pltpu.sync_copy(x_hbm.at[pl.ds(glob0, SC_BLK * D)], x_v)
        pltpu.sync_copy(mask_hbm.at[pl.ds(glob0, SC_BLK * D)], mask_v)

        def zero_body(t, carry):
            buf_v[t, pl.ds(0, 16)] = jnp.zeros((16,), jnp.float32)
            buf_v[t, pl.ds(16, 16)] = jnp.zeros((16,), jnp.float32)
            return carry

        lax.fori_loop(0, SC_BLK, zero_body, 0)
        for c in range(N_CHUNKS):
            pltpu.async_copy(emb_hbm.at[idx_v.at[c]], buf_v, sem, add=True)
        for c in range(N_CHUNKS):
            pltpu.make_async_copy(emb_hbm.at[idx_v.at[c]], buf_v, sem).wait()

        def tok_body(t, acc):
            q0 = buf_v[t, pl.ds(0, 16)] / jval
            q1 = buf_v[t, pl.ds(16, 16)] / jval
            x0 = x_v[pl.ds(t * D, 16)]
            x1 = x_v[pl.ds(t * D + 16, 16)]
            out_v[pl.ds(t * D, 16)] = x0 + (q0 - x0)
            out_v[pl.ds(t * D + 16, 16)] = x1 + (q1 - x1)
            m0 = mask_v[pl.ds(t * D, 16)]
            m1 = mask_v[pl.ds(t * D + 16, 16)]
            d0 = q0 * m0 - x0 * m0
            d1 = q1 * m1 - x1 * m1
            return acc + d0 * d0 + d1 * d1

        lacc = lax.fori_loop(0, SC_BLK, tok_body, lacc)
        pltpu.sync_copy(out_v, q_hbm.at[pl.ds(tok0 * D, SC_BLK * D)])

    lacc_v[...] = lacc
    pltpu.sync_copy(lacc_v, part_hbm.at[wid])


@functools.lru_cache(maxsize=8)
def _build_sc_gather(tok_off):
  return pl.kernel(
    functools.partial(_sc_gather_body, tok_off),
    out_type=[
        jax.ShapeDtypeStruct((SEG * D,), jnp.float32),
        jax.ShapeDtypeStruct((NW, 16), jnp.float32),
    ],
    mesh=plsc.VectorSubcoreMesh(core_axis_name="c", subcore_axis_name="s",
                                num_cores=NC, num_subcores=NS),
    compiler_params=pltpu.CompilerParams(use_tc_tiling_on_sc=False),
    scratch_types=[
        pltpu.VMEM((N_CHUNKS, SC_BLK), jnp.int32),
        pltpu.VMEM((SC_BLK, D), jnp.float32),
        pltpu.VMEM((SC_BLK * D,), jnp.float32),
        pltpu.VMEM((SC_BLK * D,), jnp.float32),
        pltpu.VMEM((SC_BLK * D,), jnp.float32),
        pltpu.VMEM((16,), jnp.float32),
        pltpu.VMEM((16,), jnp.float32),
        pltpu.SemaphoreType.DMA,
    ],
  )


def kernel(inputs_embeds, attention_mask, lab_seq_len, embedding_weight):
    initial_shape = inputs_embeds.shape
    x = inputs_embeds.reshape(-1, D)
    x_flat = inputs_embeds.reshape(-1)
    mask_flat = jnp.broadcast_to(attention_mask.reshape(-1)[:, None],
                                 (N_TOK, D)).reshape(-1)
    cb = embedding_weight[EMB_OFF:EMB_OFF + N_CHUNKS * CHUNK, :]

    j = jnp.asarray(lab_seq_len - 1, jnp.float32)
    jvec = jnp.full((16,), 1.0, jnp.float32) * j

    qs, ps = [], []
    for seg in range(NSEG):
        idx_s = _make_tc_argmin(seg * (SEG // T_BLK))(x, cb)
        q_s, p_s = _build_sc_gather(seg * SEG)(idx_s, x_flat, mask_flat,
                                               jvec, embedding_weight)
        qs.append(q_s)
        ps.append(p_s)
    q = jnp.concatenate(qs)

    e_latent = sum(jnp.sum(p) for p in ps) / jnp.float32(x_flat.size)
    loss = e_latent + 0.25 * e_latent
    return q.reshape(initial_shape), loss


# speedup vs baseline: 1.0009x; 1.0009x over previous
import functools

import jax
import jax.numpy as jnp
from jax import lax
from jax.experimental import pallas as pl
from jax.experimental.pallas import tpu as pltpu
from jax.experimental.pallas import tpu_sc as plsc

D = 32
N_CHUNKS = 16
CHUNK = 512
EMB_OFF = 2
N_TOK = 16 * 1024
NSEG = 4
SEG = N_TOK // NSEG
T_BLK = 512
NC, NS = 2, 16
NW = NC * NS
TOK_PER_TILE = SEG // NW
SC_BLK = 128
N_SC_BLKS = TOK_PER_TILE // SC_BLK


def _tc_argmin_body(x_ref, cb_ref, idx_ref):
    x = x_ref[...]
    xn = jnp.sum(x * x, axis=1, keepdims=True)
    x2 = x + x
    cols = []
    for c in range(N_CHUNKS):
        cb = cb_ref[c * CHUNK:(c + 1) * CHUNK, :]
        cn = jnp.sum(cb * cb, axis=1)[None, :]
        s2 = lax.dot_general(x2, cb, (((1,), (1,)), ((), ())),
                             preferred_element_type=jnp.float32)
        dist = (xn + cn) - s2
        dmin = jnp.min(dist, axis=1, keepdims=True)
        ii = lax.broadcasted_iota(jnp.int32, (1, CHUNK), 1).astype(jnp.float32)
        first_min = jnp.min(jnp.where(dist <= dmin, ii, jnp.float32(CHUNK)),
                            axis=1).astype(jnp.int32)
        cols.append(first_min + (EMB_OFF + c * CHUNK))
    idx_ref[...] = jnp.stack(cols, axis=0)


def _make_tc_argmin(block_off):
    return pl.pallas_call(
        _tc_argmin_body,
        grid=(SEG // T_BLK,),
        in_specs=[
            pl.BlockSpec((T_BLK, D), lambda i: (i + block_off, 0)),
            pl.BlockSpec((N_CHUNKS * CHUNK, D), lambda i: (0, 0)),
        ],
        out_specs=pl.BlockSpec((N_CHUNKS, T_BLK), lambda i: (0, i)),
        out_shape=jax.ShapeDtypeStruct((N_CHUNKS, SEG), jnp.int32),
    )


def _sc_gather_body(tok_off, idx_hbm, x_hbm, mask_hbm, jvec_hbm, emb_hbm,
                    q_hbm, part_hbm,
                    idx_v, buf_v, x_v, mask_v, out_v, jv_v, lacc_v, sem):
    cid = lax.axis_index("c")
    sid = lax.axis_index("s")
    wid = sid * NC + cid
    base = wid * TOK_PER_TILE

    pltpu.sync_copy(jvec_hbm, jv_v)
    jval = jv_v[...]
    lacc = jnp.zeros((16,), jnp.float32)

    for b in range(N_SC_BLKS):
        tok0 = base + b * SC_BLK
        glob0 = (tok_off + tok0) * D
        pltpu.sync_copy(idx_hbm.at[:, pl.ds(tok0, SC_BLK)], idx_v)
        pltpu.sync_copy(x_hbm.at[pl.ds(glob0, SC_BLK * D)], x_v)
        pltpu.sync_copy(mask_hbm.at[pl.ds(glob0, SC_BLK * D)], mask_v)

        def zero_body(t, carry):
            buf_v[t, pl.ds(0, 16)] = jnp.zeros((16,), jnp.float32)
            buf_v[t, pl.ds(16, 16)] = jnp.zeros((16,), jnp.float32)
            return carry

        lax.fori_loop(0, SC_BLK, zero_body, 0)
        for c in range(N_CHUNKS):
            pltpu.async_copy(emb_hbm.at[idx_v.at[c]], buf_v, sem, add=True)
        for c in range(N_CHUNKS):
            pltpu.make_async_copy(emb_hbm.at[idx_v.at[c]], buf_v, sem).wait()

        def tok_body(t, acc):
            q0 = buf_v[t, pl.ds(0, 16)] / jval
            q1 = buf_v[t, pl.ds(16, 16)] / jval
            x0 = x_v[pl.ds(t * D, 16)]
            x1 = x_v[pl.ds(t * D + 16, 16)]
            out_v[pl.ds(t * D, 16)] = x0 + (q0 - x0)
            out_v[pl.ds(t * D + 16, 16)] = x1 + (q1 - x1)
            m0 = mask_v[pl.ds(t * D, 16)]
            m1 = mask_v[pl.ds(t * D + 16, 16)]
            d0 = q0 * m0 - x0 * m0
            d1 = q1 * m1 - x1 * m1
            return acc + d0 * d0 + d1 * d1

        lacc = lax.fori_loop(0, SC_BLK, tok_body, lacc)
        pltpu.sync_copy(out_v, q_hbm.at[pl.ds(tok0 * D, SC_BLK * D)])

    lacc_v[...] = lacc
    pltpu.sync_copy(lacc_v, part_hbm.at[wid])


@functools.lru_cache(maxsize=8)
def _build_sc_gather(tok_off):
  return pl.kernel(
    functools.partial(_sc_gather_body, tok_off),
    out_type=[
        jax.ShapeDtypeStruct((SEG * D,), jnp.float32),
        jax.ShapeDtypeStruct((NW, 16), jnp.float32),
    ],
    mesh=plsc.VectorSubcoreMesh(core_axis_name="c", subcore_axis_name="s",
                                num_cores=NC, num_subcores=NS),
    compiler_params=pltpu.CompilerParams(use_tc_tiling_on_sc=False),
    scratch_types=[
        pltpu.VMEM((N_CHUNKS, SC_BLK), jnp.int32),
        pltpu.VMEM((SC_BLK, D), jnp.float32),
        pltpu.VMEM((SC_BLK * D,), jnp.float32),
        pltpu.VMEM((SC_BLK * D,), jnp.float32),
        pltpu.VMEM((SC_BLK * D,), jnp.float32),
        pltpu.VMEM((16,), jnp.float32),
        pltpu.VMEM((16,), jnp.float32),
        pltpu.SemaphoreType.DMA,
    ],
  )


def kernel(inputs_embeds, attention_mask, lab_seq_len, embedding_weight):
    initial_shape = inputs_embeds.shape
    x = inputs_embeds.reshape(-1, D)
    x_flat = inputs_embeds.reshape(-1)
    mask_flat = jnp.broadcast_to(attention_mask.reshape(-1)[:, None],
                                 (N_TOK, D)).reshape(-1)
    cb = embedding_weight[EMB_OFF:EMB_OFF + N_CHUNKS * CHUNK, :]

    j = jnp.asarray(lab_seq_len - 1, jnp.float32)
    jvec = jnp.full((16,), 1.0, jnp.float32) * j

    qs, ps = [], []
    for seg in range(NSEG):
        idx_s = _make_tc_argmin(seg * (SEG // T_BLK))(x, cb)
        q_s, p_s = _build_sc_gather(seg * SEG)(idx_s, x_flat, mask_flat,
                                               jvec, embedding_weight)
        qs.append(q_s)
        ps.append(p_s)
    q = jnp.concatenate(qs)

    e_latent = sum(jnp.sum(p) for p in ps) / jnp.float32(x_flat.size)
    loss = e_latent + 0.25 * e_latent
    return q.reshape(initial_shape), loss
